# baseline (speedup 1.0000x reference)
import jax
import jax.numpy as jnp
from jax.experimental import pallas as pl
from jax.experimental.pallas import tpu as pltpu

SEQ = 1024
KC = 128
VOC = 2 * KC + 1
PADL = SEQ - KC
DIM = 64
BLOCK = 32


def _body(tk_ref, tv_ref, ok_ref, ov_ref, fk8, fv8, sem_k, sem_v):
    pid = pl.program_id(0)

    @pl.when(pid == 0)
    def _build():
        for t_ref, f8 in ((tk_ref, fk8), (tv_ref, fv8)):
            for p in range(8):
                f8[p, 0:PADL - p, :] = jnp.broadcast_to(
                    t_ref[0:1, :], (PADL - p, DIM))
                f8[p, PADL - p:PADL - p + VOC, :] = t_ref[...]
                f8[p, PADL - p + VOC:2 * SEQ, :] = jnp.broadcast_to(
                    t_ref[VOC - 1:VOC, :], (2 * SEQ - PADL + p - VOC, DIM))

    for r in range(BLOCK):
        i = pid * BLOCK + r
        p = (8 - r % 8) % 8
        base = SEQ - pid * BLOCK - r - p
        pltpu.make_async_copy(
            fk8.at[p, pl.ds(base, SEQ), :], ok_ref.at[i], sem_k.at[r]).start()
        pltpu.make_async_copy(
            fv8.at[p, pl.ds(base, SEQ), :], ov_ref.at[i], sem_v.at[r]).start()
    for r in range(BLOCK):
        i = pid * BLOCK + r
        p = (8 - r % 8) % 8
        base = SEQ - pid * BLOCK - r - p
        pltpu.make_async_copy(
            fk8.at[p, pl.ds(base, SEQ), :], ok_ref.at[i], sem_k.at[r]).wait()
        pltpu.make_async_copy(
            fv8.at[p, pl.ds(base, SEQ), :], ov_ref.at[i], sem_v.at[r]).wait()


def kernel(seq_len, table_k, table_v):
    del seq_len
    out = pl.pallas_call(
        _body,
        grid=(SEQ // BLOCK,),
        in_specs=[
            pl.BlockSpec((VOC, DIM), lambda b: (0, 0)),
            pl.BlockSpec((VOC, DIM), lambda b: (0, 0)),
        ],
        out_specs=[
            pl.BlockSpec(memory_space=pl.ANY),
            pl.BlockSpec(memory_space=pl.ANY),
        ],
        out_shape=[
            jax.ShapeDtypeStruct((SEQ, SEQ, DIM), jnp.float32),
            jax.ShapeDtypeStruct((SEQ, SEQ, DIM), jnp.float32),
        ],
        scratch_shapes=[
            pltpu.VMEM((8, 2 * SEQ, DIM), jnp.float32),
            pltpu.VMEM((8, 2 * SEQ, DIM), jnp.float32),
            pltpu.SemaphoreType.DMA((BLOCK,)),
            pltpu.SemaphoreType.DMA((BLOCK,)),
        ],
    )(table_k, table_v)
    return (out[0], out[1])


# speedup vs baseline: 6.1527x; 6.1527x over previous
import jax
import jax.numpy as jnp
from jax.experimental import pallas as pl
from jax.experimental.pallas import tpu as pltpu

SEQ = 1024
KC = 128
VOC = 2 * KC + 1
DIM = 64
TB = 128
NT = SEQ // TB


def _body(tkT, tvT, ok, ov, xk, xv, ck, cv):
    bi = pl.program_id(0)
    bj = pl.program_id(1)

    @pl.when((bi == 0) & (bj == 0))
    def _build():
        for tT, x, c in ((tkT, xk, ck), (tvT, xv, cv)):
            t0 = tT[:, 0:1]
            t2 = tT[:, VOC - 1:VOC]
            c[0] = jnp.broadcast_to(t0, (DIM, TB))
            c[1] = jnp.broadcast_to(t2, (DIM, TB))
            for ii in range(TB):
                x[0, ii] = tT[:, KC - ii:KC - ii + TB]
                if ii == 0:
                    x[1, ii] = jnp.broadcast_to(t2, (DIM, TB))
                else:
                    x[1, ii] = jnp.concatenate(
                        [tT[:, 2 * KC - ii:2 * KC],
                         jnp.broadcast_to(t2, (DIM, TB - ii))], axis=1)
                if ii == TB - 1:
                    x[2, ii] = jnp.broadcast_to(t0, (DIM, TB))
                else:
                    x[2, ii] = jnp.concatenate(
                        [jnp.broadcast_to(t0, (DIM, ii + 1)),
                         tT[:, 1:TB - ii]], axis=1)

    d = bj - bi
    for out_ref, x, c in ((ok, xk, ck), (ov, xv, cv)):
        @pl.when(d == 0)
        def _():
            out_ref[...] = x[0]

        @pl.when(d == 1)
        def _():
            out_ref[...] = x[1]

        @pl.when(d == -1)
        def _():
            out_ref[...] = x[2]

        @pl.when(d >= 2)
        def _():
            out_ref[...] = jnp.broadcast_to(c[1][None], (TB, DIM, TB))

        @pl.when(d <= -2)
        def _():
            out_ref[...] = jnp.broadcast_to(c[0][None], (TB, DIM, TB))


def kernel(seq_len, table_k, table_v):
    del seq_len
    out = pl.pallas_call(
        _body,
        grid=(NT, NT),
        in_specs=[
            pl.BlockSpec((DIM, VOC), lambda bi, bj: (0, 0)),
            pl.BlockSpec((DIM, VOC), lambda bi, bj: (0, 0)),
        ],
        out_specs=[
            pl.BlockSpec((TB, DIM, TB), lambda bi, bj: (bi, 0, bj)),
            pl.BlockSpec((TB, DIM, TB), lambda bi, bj: (bi, 0, bj)),
        ],
        out_shape=[
            jax.ShapeDtypeStruct((SEQ, DIM, SEQ), jnp.float32),
            jax.ShapeDtypeStruct((SEQ, DIM, SEQ), jnp.float32),
        ],
        scratch_shapes=[
            pltpu.VMEM((3, TB, DIM, TB), jnp.float32),
            pltpu.VMEM((3, TB, DIM, TB), jnp.float32),
            pltpu.VMEM((2, DIM, TB), jnp.float32),
            pltpu.VMEM((2, DIM, TB), jnp.float32),
        ],
    )(table_k.T, table_v.T)
    return (jnp.swapaxes(out[0], 1, 2), jnp.swapaxes(out[1], 1, 2))
